# Initial kernel scaffold; baseline (speedup 1.0000x reference)
import jax
import jax.numpy as jnp
from jax.experimental import pallas as pl


def kernel(token_ids, weight):
    raise NotImplementedError("write your pallas kernel here")


# speedup vs baseline: 1.0940x; 1.0940x over previous
import functools

import jax
import jax.numpy as jnp
from jax import lax
from jax.experimental import pallas as pl
from jax.experimental.pallas import tpu as pltpu
from jax.experimental.pallas import tpu_sc as plsc

NUM_ROWS = 16384
SEQ = 50
DIM = 32
B = NUM_ROWS * SEQ

_info = plsc.get_sparse_core_info()
NC = _info.num_cores
NS = _info.num_subcores
NW = NC * NS
BPW = B // NW
CHUNK = 1024
NCHUNK = BPW // CHUNK

_mesh = plsc.VectorSubcoreMesh(core_axis_name="c", subcore_axis_name="s")


@functools.partial(
    pl.kernel,
    mesh=_mesh,
    out_type=jax.ShapeDtypeStruct((B, DIM), jnp.float32),
    scratch_types=[
        pltpu.VMEM((CHUNK,), jnp.int32),
        pltpu.VMEM((CHUNK, DIM), jnp.float32),
        pltpu.SemaphoreType.DMA,
    ],
    compiler_params=pltpu.CompilerParams(use_tc_tiling_on_sc=False),
)
def _embed_sc(idx_hbm, tbl_hbm, out_hbm, idx_v, rows_v, sem):
    wid = lax.axis_index("s") * NC + lax.axis_index("c")
    base = wid * BPW

    def body(i, carry):
        off = base + i * CHUNK
        pltpu.sync_copy(idx_hbm.at[pl.ds(off, CHUNK)], idx_v)
        pltpu.async_copy(tbl_hbm.at[idx_v], rows_v, sem).wait()
        pltpu.sync_copy(rows_v, out_hbm.at[pl.ds(off, CHUNK)])
        return carry

    lax.fori_loop(0, NCHUNK, body, 0)


def kernel(token_ids, weight):
    flat = token_ids.reshape(B)
    out = _embed_sc(flat, weight)
    return out.reshape(NUM_ROWS, SEQ, DIM)


# speedup vs baseline: 1.1129x; 1.0173x over previous
import functools

import jax
import jax.numpy as jnp
from jax import lax
from jax.experimental import pallas as pl
from jax.experimental.pallas import tpu as pltpu
from jax.experimental.pallas import tpu_sc as plsc

NUM_ROWS = 16384
SEQ = 50
DIM = 32
B = NUM_ROWS * SEQ

_info = plsc.get_sparse_core_info()
NC = _info.num_cores
NS = _info.num_subcores
NW = NC * NS
BPW = B // NW
CHUNK = 1024
NCHUNK = BPW // CHUNK
BUFS = 3
LEAD = 2

_mesh = plsc.VectorSubcoreMesh(core_axis_name="c", subcore_axis_name="s")


@functools.partial(
    pl.kernel,
    mesh=_mesh,
    out_type=jax.ShapeDtypeStruct((B, DIM), jnp.float32),
    scratch_types=[
        pltpu.VMEM((BPW,), jnp.int32),
        pltpu.VMEM((BUFS, CHUNK, DIM), jnp.float32),
        pltpu.SemaphoreType.DMA,
        pltpu.SemaphoreType.DMA,
        pltpu.SemaphoreType.DMA,
        pltpu.SemaphoreType.DMA,
        pltpu.SemaphoreType.DMA,
        pltpu.SemaphoreType.DMA,
    ],
    compiler_params=pltpu.CompilerParams(use_tc_tiling_on_sc=False),
)
def _embed_sc(idx_hbm, tbl_hbm, out_hbm, idx_v, rows_v,
              g0, g1, g2, w0, w1, w2):
    wid = lax.axis_index("s") * NC + lax.axis_index("c")
    base = wid * BPW
    gsem = (g0, g1, g2)
    wsem = (w0, w1, w2)

    pltpu.sync_copy(idx_hbm.at[pl.ds(base, BPW)], idx_v)

    def gather(k):
        b = k % BUFS
        return pltpu.make_async_copy(
            tbl_hbm.at[idx_v.at[pl.ds(k * CHUNK, CHUNK)]],
            rows_v.at[b], gsem[b])

    def write(k):
        b = k % BUFS
        return pltpu.make_async_copy(
            rows_v.at[b], out_hbm.at[pl.ds(base + k * CHUNK, CHUNK)],
            wsem[b])

    for k in range(LEAD):
        gather(k).start()
    for k in range(NCHUNK):
        nxt = k + LEAD
        if nxt < NCHUNK:
            if nxt >= BUFS:
                write(nxt - BUFS).wait()
            gather(nxt).start()
        gather(k).wait()
        write(k).start()
    for k in range(NCHUNK - BUFS, NCHUNK):
        write(k).wait()


def kernel(token_ids, weight):
    flat = token_ids.reshape(B)
    out = _embed_sc(flat, weight)
    return out.reshape(NUM_ROWS, SEQ, DIM)


# speedup vs baseline: 1.7828x; 1.6019x over previous
import functools

import jax
import jax.numpy as jnp
from jax import lax
from jax.experimental import pallas as pl
from jax.experimental.pallas import tpu as pltpu
from jax.experimental.pallas import tpu_sc as plsc

NUM_ROWS = 16384
SEQ = 50
DIM = 32

_info = plsc.get_sparse_core_info()
NC = _info.num_cores
NS = _info.num_subcores
NW = NC * NS
RPW = NUM_ROWS // NW
CROWS = 16
NCHUNK = RPW // CROWS
BUFS = 3
LEAD = 2

_mesh = plsc.VectorSubcoreMesh(core_axis_name="c", subcore_axis_name="s")


@functools.partial(
    pl.kernel,
    mesh=_mesh,
    out_type=jax.ShapeDtypeStruct((NUM_ROWS, SEQ, DIM), jnp.float32),
    scratch_types=[
        pltpu.VMEM((BUFS, CROWS, SEQ), jnp.int32),
        pltpu.VMEM((BUFS, CROWS, SEQ, DIM), jnp.float32),
        pltpu.SemaphoreType.DMA,
        pltpu.SemaphoreType.DMA,
        pltpu.SemaphoreType.DMA,
        pltpu.SemaphoreType.DMA,
        pltpu.SemaphoreType.DMA,
        pltpu.SemaphoreType.DMA,
    ],
    compiler_params=pltpu.CompilerParams(use_tc_tiling_on_sc=False),
)
def _embed_sc(idx_hbm, tbl_hbm, out_hbm, idx_v, rows_v,
              g0, g1, g2, w0, w1, w2):
    wid = lax.axis_index("s") * NC + lax.axis_index("c")
    base = wid * RPW
    gsem = (g0, g1, g2)
    wsem = (w0, w1, w2)

    def gather(k):
        b = k % BUFS
        return pltpu.make_async_copy(
            out_hbm.at[pl.ds(base + k * CROWS, CROWS)], rows_v.at[b],
            gsem[b])

    def gather_start(k):
        b = k % BUFS
        pltpu.sync_copy(idx_hbm.at[pl.ds(base + k * CROWS, CROWS)],
                        idx_v.at[b])
        for j in range(CROWS):
            pltpu.make_async_copy(
                tbl_hbm.at[idx_v.at[b, j]], rows_v.at[b, j],
                gsem[b]).start()

    def write(k):
        b = k % BUFS
        return pltpu.make_async_copy(
            rows_v.at[b], out_hbm.at[pl.ds(base + k * CROWS, CROWS)],
            wsem[b])

    for k in range(LEAD):
        gather_start(k)
    for k in range(NCHUNK):
        nxt = k + LEAD
        if nxt < NCHUNK:
            if nxt >= BUFS:
                write(nxt - BUFS).wait()
            gather_start(nxt)
        gather(k).wait()
        write(k).start()
    for k in range(NCHUNK - BUFS, NCHUNK):
        write(k).wait()


def kernel(token_ids, weight):
    return _embed_sc(token_ids, weight)
